# baseline (speedup 1.0000x reference)
import functools

import jax
import jax.numpy as jnp
from jax.experimental import pallas as pl
import jax.experimental.pallas.tpu as pltpu

_ANCHOR_W = (10.0, 16.0, 33.0)
_N_CLS = 80
_NCH = 5 + _N_CLS

_TILE = 8192
_NBUF = 4


def _decode_body(stride_ref, x_hbm, o_hbm, in_buf, out_buf, in_sems, out_sems,
                 *, n_steps, n_tiles, n_a, n_g):
    stride = stride_ref[0]
    tile = _TILE

    def in_copy(s):
        slot = jax.lax.rem(s, _NBUF)
        idx = s // n_tiles
        ts = jax.lax.rem(s, n_tiles)
        return pltpu.make_async_copy(
            x_hbm.at[idx, :, pl.ds(ts * tile, tile)],
            in_buf.at[slot],
            in_sems.at[slot],
        )

    def out_copy(s):
        slot = jax.lax.rem(s, _NBUF)
        idx = s // n_tiles
        ts = jax.lax.rem(s, n_tiles)
        return pltpu.make_async_copy(
            out_buf.at[slot],
            o_hbm.at[idx, pl.ds(ts * tile, tile)],
            out_sems.at[slot],
        )

    for i in range(_NBUF):
        in_copy(jnp.int32(i)).start()

    def step(s, carry):
        slot = jax.lax.rem(s, _NBUF)
        idx = s // n_tiles
        ts = jax.lax.rem(s, n_tiles)
        a = jax.lax.rem(idx, n_a)
        aw = jnp.where(a == 0, _ANCHOR_W[0],
                       jnp.where(a == 1, _ANCHOR_W[1], _ANCHOR_W[2]))

        in_copy(s).wait()

        @pl.when(s >= _NBUF)
        def _():
            out_copy(s - _NBUF).wait()

        x = in_buf[slot]
        ltrb = jnp.exp(x[0:4, :]) * (aw / stride)
        l = ltrb[0:1, :]
        tt = ltrb[1:2, :]
        r = ltrb[2:3, :]
        b = ltrb[3:4, :]
        hw = ts * tile + jax.lax.broadcasted_iota(jnp.int32, (1, tile), 1)
        gx = (hw % n_g).astype(jnp.float32) + 0.5
        gy = (hw // n_g).astype(jnp.float32) + 0.5
        xc = (gx + (r - l) * 0.5) * stride
        yc = (gy + (b - tt) * 0.5) * stride
        w = (l + r) * stride
        h = (tt + b) * stride
        sig = jax.nn.sigmoid(x[4:_NCH, :])
        out = jnp.concatenate([xc, yc, w, h, sig], axis=0)
        out_buf[slot] = out.T

        out_copy(s).start()

        @pl.when(s + _NBUF < n_steps)
        def _():
            in_copy(s + _NBUF).start()

        return carry

    jax.lax.fori_loop(0, n_steps, step, 0)

    for i in range(_NBUF):
        s = jnp.int32(n_steps - _NBUF + i)
        out_copy(s).wait()


def kernel(raw, img_size):
    n_b = raw.shape[0]
    n_g = raw.shape[2]
    n_a = raw.shape[1] // _NCH
    n_hw = n_g * n_g
    stride = jnp.asarray(img_size // n_g, jnp.float32).reshape(1)

    n_tiles = n_hw // _TILE
    n_steps = n_b * n_a * n_tiles
    rr = raw.reshape(n_b * n_a, _NCH, n_hw)

    out = pl.pallas_call(
        functools.partial(_decode_body, n_steps=n_steps, n_tiles=n_tiles,
                          n_a=n_a, n_g=n_g),
        in_specs=[
            pl.BlockSpec(memory_space=pltpu.SMEM),
            pl.BlockSpec(memory_space=pl.ANY),
        ],
        out_specs=pl.BlockSpec(memory_space=pl.ANY),
        out_shape=jax.ShapeDtypeStruct((n_b * n_a, n_hw, _NCH), jnp.float32),
        scratch_shapes=[
            pltpu.VMEM((_NBUF, _NCH, _TILE), jnp.float32),
            pltpu.VMEM((_NBUF, _TILE, _NCH), jnp.float32),
            pltpu.SemaphoreType.DMA((_NBUF,)),
            pltpu.SemaphoreType.DMA((_NBUF,)),
        ],
    )(stride, rr)
    return out.reshape(n_b, n_a * n_hw, _NCH)
